# baseline (speedup 1.0000x reference)
import functools

import jax
import jax.numpy as jnp
from jax import lax
from jax.experimental import pallas as pl
from jax.experimental.pallas import tpu as pltpu
from jax.experimental.pallas import tpu_sc as plsc

_NC, _NS, _LANES = 2, 16, 16


def kernel(x, row_w, col_w):
    n, dim, h, w = x.shape
    half = dim // 2
    hw = h * w
    nw = _NC * _NS
    rows_per = dim // nw
    mesh = plsc.VectorSubcoreMesh(
        core_axis_name="c", subcore_axis_name="s",
        num_cores=_NC, num_subcores=_NS,
    )

    @functools.partial(
        pl.kernel,
        out_type=jax.ShapeDtypeStruct((n, dim, hw), jnp.float32),
        mesh=mesh,
        scratch_types=[
            pltpu.VMEM((h, half), jnp.float32),
            pltpu.VMEM((rows_per, hw), jnp.float32),
            pltpu.SemaphoreType.DMA,
        ],
        compiler_params=pltpu.CompilerParams(needs_layout_passes=False),
    )
    def pos_sc(row_hbm, col_hbm, out_hbm, tab_v, rows_v, sem):
        cid = lax.axis_index("c")
        sid = lax.axis_index("s")
        wid = cid * _NS + sid
        d_base = wid * rows_per
        is_col = wid < (nw // 2)

        @pl.when(is_col)
        def _():
            pltpu.sync_copy(col_hbm, tab_v)

        @pl.when(jnp.logical_not(is_col))
        def _():
            pltpu.sync_copy(row_hbm, tab_v)

        dloc0 = d_base - jnp.where(is_col, 0, half)
        iota = lax.iota(jnp.int32, _LANES)

        @pl.when(is_col)
        def _():
            for k in range(rows_per):
                dvec = jnp.full((_LANES,), dloc0 + k, jnp.int32)
                c_parts = [
                    plsc.load_gather(tab_v, [iota + t * _LANES, dvec])
                    for t in range(w // _LANES)
                ]
                for m in range(hw // _LANES):
                    rows_v[k, pl.ds(m * _LANES, _LANES)] = (
                        c_parts[m % (w // _LANES)]
                    )

        @pl.when(jnp.logical_not(is_col))
        def _():
            for k in range(rows_per):
                dvec = jnp.full((_LANES,), dloc0 + k, jnp.int32)
                for g in range(h):
                    gvec = jnp.full((_LANES,), g, jnp.int32)
                    val = plsc.load_gather(tab_v, [gvec, dvec])
                    for t in range(w // _LANES):
                        rows_v[k, pl.ds(g * w + t * _LANES, _LANES)] = val

        copies = []
        for nn in range(n):
            copies.append(
                pltpu.async_copy(
                    rows_v, out_hbm.at[nn, pl.ds(d_base, rows_per)], sem
                )
            )
        for cp in copies:
            cp.wait()

    out = pos_sc(row_w[:h], col_w[:w])
    return out.reshape(n, dim, h, w)
